# baseline (speedup 1.0000x reference)
import functools

import jax
import jax.numpy as jnp
from jax import lax
from jax.experimental import pallas as pl
from jax.experimental.pallas import tpu as pltpu
from jax.experimental.pallas import tpu_sc as plsc

_N = 20000
_C = 80
_M = 64
_IMG = 640.0
_RADIUS = 0.75
_SIGMA = 0.5
_MARGIN = 8.0
_AREA_T = 0.8
_ALPHA = 0.75
_GAMMA = 2.0

def _axis_index(name, dim):
    return lax.axis_index(name)


_NW = 32
_LANES = 16
_CHUNKS = _N // _LANES
_BASE_CHUNKS = _CHUNKS // _NW
_EXTRA = _CHUNKS - _BASE_CHUNKS * _NW
_MAXC = _BASE_CHUNKS + 1
_LOC = _MAXC * _LANES


def _sc_assign_body(xs_hbm, ys_hbm, gx1_hbm, gy1_hbm, gx2_hbm, gy2_hbm, w_hbm,
                    heat_hbm, rwc_hbm, anyc_hbm,
                    xs_v, ys_v, gx1_v, gy1_v, gx2_v, gy2_v, w_v,
                    heat_v, rwc_v, anyc_v):
    cid = _axis_index("c", 0)
    sid = _axis_index("s", 1)
    wid = sid * 2 + cid
    cnt = _BASE_CHUNKS + jnp.where(wid >= _NW - _EXTRA, 1, 0)
    start = _BASE_CHUNKS * wid + jnp.maximum(wid - (_NW - _EXTRA), 0)
    row0 = start * _LANES

    pltpu.sync_copy(xs_hbm.at[pl.ds(row0, _LOC)], xs_v)
    pltpu.sync_copy(ys_hbm.at[pl.ds(row0, _LOC)], ys_v)
    pltpu.sync_copy(gx1_hbm, gx1_v)
    pltpu.sync_copy(gy1_hbm, gy1_v)
    pltpu.sync_copy(gx2_hbm, gx2_v)
    pltpu.sync_copy(gy2_hbm, gy2_v)
    pltpu.sync_copy(w_hbm, w_v)

    bx1, by1, bx2, by2, bcx, bcy, bisx, bisy, bw = [], [], [], [], [], [], [], [], []
    for j in range(_M // _LANES):
        s = pl.ds(j * _LANES, _LANES)
        x1 = gx1_v[s]
        y1 = gy1_v[s]
        x2 = gx2_v[s]
        y2 = gy2_v[s]
        hx = jnp.maximum((x2 - x1) * 0.5, 1.0)
        hy = jnp.maximum((y2 - y1) * 0.5, 1.0)
        bx1.append(x1)
        by1.append(y1)
        bx2.append(x2)
        by2.append(y2)
        bcx.append((x1 + x2) * 0.5)
        bcy.append((y1 + y2) * 0.5)
        bisx.append(1.0 / (hx * _SIGMA))
        bisy.append(1.0 / (hy * _SIGMA))
        bw.append(w_v[s])

    lim = _RADIUS / _SIGMA

    def outer(i, carry):
        lo, hi = carry
        base = i * _LANES
        xv = xs_v[pl.ds(base, _LANES)]
        yv = ys_v[pl.ds(base, _LANES)]
        vmask = jnp.where(i < cnt, jnp.int32(-1), jnp.int32(0))
        heat = jnp.zeros((_LANES,), jnp.float32)
        rwc = jnp.zeros((_LANES,), jnp.float32)
        for j in range(_M // _LANES):
            for k in range(_LANES):
                m = j * _LANES + k
                x1 = bx1[j][k]
                y1 = by1[j][k]
                x2 = bx2[j][k]
                y2 = by2[j][k]
                dxs = (xv - bcx[j][k]) * bisx[j][k]
                dys = (yv - bcy[j][k]) * bisy[j][k]
                ins = (xv >= x1) & (xv <= x2) & (yv >= y1) & (yv <= y2)
                h = jnp.exp(-0.5 * (dxs * dxs + dys * dys))
                heat = jnp.maximum(heat, jnp.where(ins, h, 0.0))
                cand = ins & (jnp.abs(dxs) <= lim) & (jnp.abs(dys) <= lim)
                rwc = jnp.maximum(rwc, jnp.where(cand, bw[j][k], 0.0))
                mbit = (1 << (m % 32)) & 0xFFFFFFFF
                mbit = mbit - (1 << 32) if mbit >= (1 << 31) else mbit
                hit = jnp.where(cand, jnp.int32(mbit), 0) & vmask
                if m < 32:
                    lo = lo | hit
                else:
                    hi = hi | hit
        heat_v[pl.ds(base, _LANES)] = heat
        rwc_v[pl.ds(base, _LANES)] = rwc
        return lo, hi

    zi = jnp.zeros((_LANES,), jnp.int32)
    lo, hi = lax.fori_loop(0, _MAXC, outer, (zi, zi))
    anyc_v[pl.ds(0, _LANES)] = lo
    anyc_v[pl.ds(_LANES, _LANES)] = hi

    base_rows = _BASE_CHUNKS * _LANES
    pltpu.sync_copy(heat_v.at[pl.ds(0, base_rows)], heat_hbm.at[pl.ds(row0, base_rows)])
    pltpu.sync_copy(rwc_v.at[pl.ds(0, base_rows)], rwc_hbm.at[pl.ds(row0, base_rows)])

    @pl.when(cnt > _BASE_CHUNKS)
    def _():
        pltpu.sync_copy(heat_v.at[pl.ds(base_rows, _LANES)],
                        heat_hbm.at[pl.ds(row0 + base_rows, _LANES)])
        pltpu.sync_copy(rwc_v.at[pl.ds(base_rows, _LANES)],
                        rwc_hbm.at[pl.ds(row0 + base_rows, _LANES)])

    pltpu.sync_copy(anyc_v, anyc_hbm.at[wid])


def _sc_assign(xs, ys, gx1, gy1, gx2, gy2, w):
    call = functools.partial(
        pl.kernel,
        out_type=(
            jax.ShapeDtypeStruct((_N,), jnp.float32),
            jax.ShapeDtypeStruct((_N,), jnp.float32),
            jax.ShapeDtypeStruct((_NW, 2 * _LANES), jnp.int32),
        ),
        mesh=plsc.VectorSubcoreMesh(core_axis_name="c", subcore_axis_name="s"),
        scratch_types=[
            pltpu.VMEM((_LOC,), jnp.float32),
            pltpu.VMEM((_LOC,), jnp.float32),
            pltpu.VMEM((_M,), jnp.float32),
            pltpu.VMEM((_M,), jnp.float32),
            pltpu.VMEM((_M,), jnp.float32),
            pltpu.VMEM((_M,), jnp.float32),
            pltpu.VMEM((_M,), jnp.float32),
            pltpu.VMEM((_LOC,), jnp.float32),
            pltpu.VMEM((_LOC,), jnp.float32),
            pltpu.VMEM((2 * _LANES,), jnp.int32),
        ],
    )(_sc_assign_body)
    return call(xs, ys, gx1, gy1, gx2, gy2, w)


def _margin_body(scores_ref, out_ref):
    mx = jnp.max(scores_ref[...], axis=1, keepdims=True)
    out_ref[0] = jax.nn.relu(mx - _MARGIN)


def _tc_margin(pred_scores):
    nb = 10
    blk = _N // nb
    out = pl.pallas_call(
        _margin_body,
        grid=(nb,),
        in_specs=[pl.BlockSpec((blk, _C), lambda i: (i, 0))],
        out_specs=pl.BlockSpec((1, blk, 1), lambda i: (i, 0, 0)),
        out_shape=jax.ShapeDtypeStruct((nb, blk, 1), jnp.float32),
    )(pred_scores)
    return out.reshape(_N)


_NB2 = 10
_BLK2 = _N // _NB2


def _combine_body(xs_ref, ys_ref, obj_ref, px1_ref, py1_ref, px2_ref, py2_ref,
                  gt_ref, w_ref, heat_ref, rwc_ref, anyc_ref, ml_ref,
                  acc_ref, out_ref, wfb_ref, hasfb_ref, rw_ref):
    i = pl.program_id(0)

    @pl.when(i == 0)
    def _():
        lohi = anyc_ref[...]
        blo = lohi[:, :_LANES]
        bhi = lohi[:, _LANES:]
        sh = lax.broadcasted_iota(jnp.int32, (1, 1, 32), 2)
        any_lo = jnp.max(jnp.max((blo[:, :, None] >> sh) & 1, axis=0), axis=0)
        any_hi = jnp.max(jnp.max((bhi[:, :, None] >> sh) & 1, axis=0), axis=0)
        anyc = jnp.concatenate([any_lo, any_hi]).astype(jnp.float32)
        wfb = jnp.where(anyc > 0.0, 0.0, w_ref[...])
        wfb_ref[...] = wfb
        hasfb_ref[0] = jnp.where(jnp.sum(wfb) > 0.0, 1, 0).astype(jnp.int32)

    has_fb = hasfb_ref[0] > 0

    @pl.when(has_fb)
    def _():
        x = xs_ref[0, 0, :]
        y = ys_ref[0, 0, :]
        b = gt_ref[...]
        x1 = b[:, 0][:, None]
        y1 = b[:, 1][:, None]
        x2 = b[:, 2][:, None]
        y2 = b[:, 3][:, None]
        xr = x[None, :]
        yr = y[None, :]
        inside = (xr >= x1) & (xr <= x2) & (yr >= y1) & (yr <= y2)
        fb = jnp.max(jnp.where(inside, wfb_ref[...][:, None], 0.0), axis=0)
        rw_ref[0, :] = jnp.maximum(rwc_ref[0, 0, :], fb)

    @pl.when(jnp.logical_not(has_fb))
    def _():
        rw_ref[0, :] = rwc_ref[0, 0, :]

    rw = rw_ref[0, :]

    heat_t = heat_ref[0, 0, :]
    l = obj_ref[0, 0, :]
    prob = jax.nn.sigmoid(l)
    vfl_w = _ALPHA * prob * prob * (1.0 - heat_t) + heat_t
    bce = jnp.maximum(l, 0.0) - l * heat_t + jnp.log1p(jnp.exp(-jnp.abs(l)))
    vfl_sum = jnp.sum(bce * vfl_w)

    m_num = jnp.sum(ml_ref[0, 0, :] * rw)
    m_den = jnp.sum(rw)

    bx1 = jnp.clip(px1_ref[0, 0, :], 0.0, _IMG)
    by1 = jnp.clip(py1_ref[0, 0, :], 0.0, _IMG)
    bx2 = jnp.clip(px2_ref[0, 0, :], 0.0, _IMG)
    by2 = jnp.clip(py2_ref[0, 0, :], 0.0, _IMG)
    wd = jnp.maximum(bx2 - bx1, 0.0)
    ht = jnp.maximum(by2 - by1, 0.0)
    area_ratio = wd * ht * (1.0 / (_IMG * _IMG))
    edge = ((bx1 <= 1.0).astype(jnp.float32) + (by1 <= 1.0).astype(jnp.float32) +
            (bx2 >= _IMG - 1.0).astype(jnp.float32) +
            (by2 >= _IMG - 1.0).astype(jnp.float32)) * 0.25
    pen_sum = jnp.sum(jnp.maximum(area_ratio - _AREA_T, 0.0) * (1.0 + edge))

    lanes = lax.broadcasted_iota(jnp.int32, (128,), 0)
    rowv = (jnp.where(lanes == 0, vfl_sum, 0.0) +
            jnp.where(lanes == 1, m_num, 0.0) +
            jnp.where(lanes == 2, m_den, 0.0) +
            jnp.where(lanes == 3, pen_sum, 0.0))

    @pl.when(i == 0)
    def _():
        acc_ref[...] = jnp.zeros((1, 128), jnp.float32)

    acc_ref[...] += rowv[None, :]

    @pl.when(i == _NB2 - 1)
    def _():
        v = acc_ref[0, :]
        total = (v[0] * (1.0 / _N) + v[1] / jnp.maximum(v[2], 1e-6)
                 + v[3] * (1.0 / _N))
        out_ref[...] = jnp.full((1, 1), total, jnp.float32)


def _tc_combine(xs, ys, obj, px1, py1, px2, py2, gt, w, heat, rwc, anyc, ml):
    blkmap3 = pl.BlockSpec((1, 1, _BLK2), lambda i: (i, 0, 0))
    full = lambda a: pl.BlockSpec(a.shape, lambda i: (0,) * a.ndim)
    r3 = lambda a: a.reshape(_NB2, 1, _BLK2)
    xs, ys, obj, px1, py1, px2, py2, heat, rwc, ml = map(
        r3, (xs, ys, obj, px1, py1, px2, py2, heat, rwc, ml))
    _, out = pl.pallas_call(
        _combine_body,
        grid=(_NB2,),
        in_specs=[blkmap3] * 7 + [full(gt), full(w), blkmap3, blkmap3,
                                  full(anyc), blkmap3],
        out_specs=(pl.BlockSpec((1, 128), lambda i: (0, 0)),
                   pl.BlockSpec((1, 1), lambda i: (0, 0))),
        out_shape=(jax.ShapeDtypeStruct((1, 128), jnp.float32),
                   jax.ShapeDtypeStruct((1, 1), jnp.float32)),
        scratch_shapes=[
            pltpu.VMEM((_M,), jnp.float32),
            pltpu.SMEM((1,), jnp.int32),
            pltpu.VMEM((1, _BLK2), jnp.float32),
        ],
    )(xs, ys, obj, px1, py1, px2, py2, gt, w, heat, rwc, anyc, ml)
    return out


def kernel(pred_scores, pred_boxes, pred_objectness, anchor_points, gt_boxes, gt_labels, box_weights):
    xs = anchor_points[:, 0]
    ys = anchor_points[:, 1]
    gx1 = gt_boxes[:, 0]
    gy1 = gt_boxes[:, 1]
    gx2 = gt_boxes[:, 2]
    gy2 = gt_boxes[:, 3]
    heat, rwc, anyc = _sc_assign(xs, ys, gx1, gy1, gx2, gy2, box_weights)
    ml = _tc_margin(pred_scores)
    out = _tc_combine(xs, ys, pred_objectness,
                      pred_boxes[:, 0], pred_boxes[:, 1],
                      pred_boxes[:, 2], pred_boxes[:, 3],
                      gt_boxes, box_weights, heat, rwc, anyc, ml)
    return out[0, 0]


# speedup vs baseline: 1.3567x; 1.3567x over previous
import functools

import jax
import jax.numpy as jnp
from jax import lax
from jax.experimental import pallas as pl
from jax.experimental.pallas import tpu as pltpu
from jax.experimental.pallas import tpu_sc as plsc

_N = 20000
_C = 80
_M = 64
_IMG = 640.0
_RADIUS = 0.75
_SIGMA = 0.5
_MARGIN = 8.0
_AREA_T = 0.8
_ALPHA = 0.75
_GAMMA = 2.0

def _axis_index(name, dim):
    return lax.axis_index(name)


_NW = 32
_LANES = 16
_CHUNKS = _N // _LANES
_BASE_CHUNKS = _CHUNKS // _NW
_EXTRA = _CHUNKS - _BASE_CHUNKS * _NW
_MAXC = _BASE_CHUNKS + 1
_LOC = _MAXC * _LANES


def _sc_assign_body(xs_hbm, ys_hbm, gx1_hbm, gy1_hbm, gx2_hbm, gy2_hbm, w_hbm,
                    heat_hbm, rwc_hbm, anyc_hbm,
                    xs_v, ys_v, gx1_v, gy1_v, gx2_v, gy2_v, w_v,
                    heat_v, rwc_v, anyc_v):
    cid = _axis_index("c", 0)
    sid = _axis_index("s", 1)
    wid = sid * 2 + cid
    cnt = _BASE_CHUNKS + jnp.where(wid >= _NW - _EXTRA, 1, 0)
    start = _BASE_CHUNKS * wid + jnp.maximum(wid - (_NW - _EXTRA), 0)
    row0 = start * _LANES

    pltpu.sync_copy(xs_hbm.at[pl.ds(row0, _LOC)], xs_v)
    pltpu.sync_copy(ys_hbm.at[pl.ds(row0, _LOC)], ys_v)
    pltpu.sync_copy(gx1_hbm, gx1_v)
    pltpu.sync_copy(gy1_hbm, gy1_v)
    pltpu.sync_copy(gx2_hbm, gx2_v)
    pltpu.sync_copy(gy2_hbm, gy2_v)
    pltpu.sync_copy(w_hbm, w_v)

    bx1, by1, bx2, by2, bcx, bcy, bisx, bisy, bw = [], [], [], [], [], [], [], [], []
    for j in range(_M // _LANES):
        s = pl.ds(j * _LANES, _LANES)
        x1 = gx1_v[s]
        y1 = gy1_v[s]
        x2 = gx2_v[s]
        y2 = gy2_v[s]
        hx = jnp.maximum((x2 - x1) * 0.5, 1.0)
        hy = jnp.maximum((y2 - y1) * 0.5, 1.0)
        bx1.append(x1)
        by1.append(y1)
        bx2.append(x2)
        by2.append(y2)
        bcx.append((x1 + x2) * 0.5)
        bcy.append((y1 + y2) * 0.5)
        bisx.append(0.7071067811865476 / (hx * _SIGMA))
        bisy.append(0.7071067811865476 / (hy * _SIGMA))
        bw.append(w_v[s])

    lim = (_RADIUS / _SIGMA) * 0.7071067811865476
    big = jnp.float32(3.0e38)

    def outer(i, carry):
        lo, hi = carry
        base = i * _LANES
        xv = xs_v[pl.ds(base, _LANES)]
        yv = ys_v[pl.ds(base, _LANES)]
        vmask = jnp.where(i < cnt, jnp.int32(-1), jnp.int32(0))
        d2m = jnp.full((_LANES,), big, jnp.float32)
        rwc = jnp.zeros((_LANES,), jnp.float32)
        loc = jnp.zeros((_LANES,), jnp.int32)
        hic = jnp.zeros((_LANES,), jnp.int32)
        for j in range(_M // _LANES):
            for k in range(_LANES):
                m = j * _LANES + k
                x1 = bx1[j][k]
                y1 = by1[j][k]
                x2 = bx2[j][k]
                y2 = by2[j][k]
                dxs = (xv - bcx[j][k]) * bisx[j][k]
                dys = (yv - bcy[j][k]) * bisy[j][k]
                ins = (xv >= x1) & (xv <= x2) & (yv >= y1) & (yv <= y2)
                d2 = dxs * dxs + dys * dys
                d2m = jnp.minimum(d2m, jnp.where(ins, d2, big))
                cand = ins & (jnp.maximum(jnp.abs(dxs), jnp.abs(dys)) <= lim)
                rwc = jnp.maximum(rwc, jnp.where(cand, bw[j][k], 0.0))
                mbit = (1 << (m % 32)) & 0xFFFFFFFF
                mbit = mbit - (1 << 32) if mbit >= (1 << 31) else mbit
                hit = jnp.where(cand, jnp.int32(mbit), 0)
                if m < 32:
                    loc = loc | hit
                else:
                    hic = hic | hit
        heat_v[pl.ds(base, _LANES)] = jnp.exp(-jnp.minimum(d2m, 100.0))
        rwc_v[pl.ds(base, _LANES)] = rwc
        return lo | (loc & vmask), hi | (hic & vmask)

    zi = jnp.zeros((_LANES,), jnp.int32)
    lo, hi = lax.fori_loop(0, _MAXC, outer, (zi, zi))
    anyc_v[pl.ds(0, _LANES)] = lo
    anyc_v[pl.ds(_LANES, _LANES)] = hi

    base_rows = _BASE_CHUNKS * _LANES
    pltpu.sync_copy(heat_v.at[pl.ds(0, base_rows)], heat_hbm.at[pl.ds(row0, base_rows)])
    pltpu.sync_copy(rwc_v.at[pl.ds(0, base_rows)], rwc_hbm.at[pl.ds(row0, base_rows)])

    @pl.when(cnt > _BASE_CHUNKS)
    def _():
        pltpu.sync_copy(heat_v.at[pl.ds(base_rows, _LANES)],
                        heat_hbm.at[pl.ds(row0 + base_rows, _LANES)])
        pltpu.sync_copy(rwc_v.at[pl.ds(base_rows, _LANES)],
                        rwc_hbm.at[pl.ds(row0 + base_rows, _LANES)])

    pltpu.sync_copy(anyc_v, anyc_hbm.at[wid])


def _sc_assign(xs, ys, gx1, gy1, gx2, gy2, w):
    call = functools.partial(
        pl.kernel,
        out_type=(
            jax.ShapeDtypeStruct((_N,), jnp.float32),
            jax.ShapeDtypeStruct((_N,), jnp.float32),
            jax.ShapeDtypeStruct((_NW, 2 * _LANES), jnp.int32),
        ),
        mesh=plsc.VectorSubcoreMesh(core_axis_name="c", subcore_axis_name="s"),
        scratch_types=[
            pltpu.VMEM((_LOC,), jnp.float32),
            pltpu.VMEM((_LOC,), jnp.float32),
            pltpu.VMEM((_M,), jnp.float32),
            pltpu.VMEM((_M,), jnp.float32),
            pltpu.VMEM((_M,), jnp.float32),
            pltpu.VMEM((_M,), jnp.float32),
            pltpu.VMEM((_M,), jnp.float32),
            pltpu.VMEM((_LOC,), jnp.float32),
            pltpu.VMEM((_LOC,), jnp.float32),
            pltpu.VMEM((2 * _LANES,), jnp.int32),
        ],
    )(_sc_assign_body)
    return call(xs, ys, gx1, gy1, gx2, gy2, w)


def _margin_body(scores_ref, out_ref):
    mx = jnp.max(scores_ref[...], axis=1, keepdims=True)
    out_ref[0] = jax.nn.relu(mx - _MARGIN)


def _tc_margin(pred_scores):
    nb = 4
    blk = _N // nb
    out = pl.pallas_call(
        _margin_body,
        grid=(nb,),
        in_specs=[pl.BlockSpec((blk, _C), lambda i: (i, 0))],
        out_specs=pl.BlockSpec((1, blk, 1), lambda i: (i, 0, 0)),
        out_shape=jax.ShapeDtypeStruct((nb, blk, 1), jnp.float32),
    )(pred_scores)
    return out.reshape(_N)


_NB2 = 10
_BLK2 = _N // _NB2


def _combine_body(xs_ref, ys_ref, obj_ref, px1_ref, py1_ref, px2_ref, py2_ref,
                  gt_ref, w_ref, heat_ref, rwc_ref, anyc_ref, ml_ref,
                  out_ref, rw_ref):
    lohi = anyc_ref[...]
    blo = lohi[:, :_LANES]
    bhi = lohi[:, _LANES:]
    sh = lax.broadcasted_iota(jnp.int32, (1, 1, 32), 2)
    any_lo = jnp.max(jnp.max((blo[:, :, None] >> sh) & 1, axis=0), axis=0)
    any_hi = jnp.max(jnp.max((bhi[:, :, None] >> sh) & 1, axis=0), axis=0)
    anyc = jnp.concatenate([any_lo, any_hi]).astype(jnp.float32)
    wfb = jnp.where(anyc > 0.0, 0.0, w_ref[...])[:, None]
    has_fb = jnp.sum(wfb) > 0.0

    b = gt_ref[...]
    x1 = b[:, 0][:, None]
    y1 = b[:, 1][:, None]
    x2 = b[:, 2][:, None]
    y2 = b[:, 3][:, None]

    @pl.when(has_fb)
    def _():
        for c in range(_NB2):
            s = pl.ds(c * _BLK2, _BLK2)
            xr = xs_ref[s][None, :]
            yr = ys_ref[s][None, :]
            inside = (xr >= x1) & (xr <= x2) & (yr >= y1) & (yr <= y2)
            fb = jnp.max(jnp.where(inside, wfb, 0.0), axis=0)
            rw_ref[s] = jnp.maximum(rwc_ref[s], fb)

    @pl.when(jnp.logical_not(has_fb))
    def _():
        rw_ref[...] = rwc_ref[...]

    rw = rw_ref[...]

    heat_t = heat_ref[...]
    l = obj_ref[...]
    prob = jax.nn.sigmoid(l)
    vfl_w = _ALPHA * prob * prob * (1.0 - heat_t) + heat_t
    bce = jnp.maximum(l, 0.0) - l * heat_t + jnp.log1p(jnp.exp(-jnp.abs(l)))
    vfl_sum = jnp.sum(bce * vfl_w)

    m_num = jnp.sum(ml_ref[...] * rw)
    m_den = jnp.maximum(jnp.sum(rw), 1e-6)

    bx1 = jnp.clip(px1_ref[...], 0.0, _IMG)
    by1 = jnp.clip(py1_ref[...], 0.0, _IMG)
    bx2 = jnp.clip(px2_ref[...], 0.0, _IMG)
    by2 = jnp.clip(py2_ref[...], 0.0, _IMG)
    wd = jnp.maximum(bx2 - bx1, 0.0)
    ht = jnp.maximum(by2 - by1, 0.0)
    area_ratio = wd * ht * (1.0 / (_IMG * _IMG))
    edge = ((bx1 <= 1.0).astype(jnp.float32) + (by1 <= 1.0).astype(jnp.float32) +
            (bx2 >= _IMG - 1.0).astype(jnp.float32) +
            (by2 >= _IMG - 1.0).astype(jnp.float32)) * 0.25
    pen_sum = jnp.sum(jnp.maximum(area_ratio - _AREA_T, 0.0) * (1.0 + edge))

    total = (vfl_sum * (1.0 / _N) + m_num / m_den + pen_sum * (1.0 / _N))
    out_ref[...] = jnp.full((1, 1), total, jnp.float32)


def _tc_combine(xs, ys, obj, px1, py1, px2, py2, gt, w, heat, rwc, anyc, ml):
    return pl.pallas_call(
        _combine_body,
        out_shape=jax.ShapeDtypeStruct((1, 1), jnp.float32),
        scratch_shapes=[pltpu.VMEM((_N,), jnp.float32)],
    )(xs, ys, obj, px1, py1, px2, py2, gt, w, heat, rwc, anyc, ml)


def kernel(pred_scores, pred_boxes, pred_objectness, anchor_points, gt_boxes, gt_labels, box_weights):
    xs = anchor_points[:, 0]
    ys = anchor_points[:, 1]
    gx1 = gt_boxes[:, 0]
    gy1 = gt_boxes[:, 1]
    gx2 = gt_boxes[:, 2]
    gy2 = gt_boxes[:, 3]
    heat, rwc, anyc = _sc_assign(xs, ys, gx1, gy1, gx2, gy2, box_weights)
    ml = _tc_margin(pred_scores)
    out = _tc_combine(xs, ys, pred_objectness,
                      pred_boxes[:, 0], pred_boxes[:, 1],
                      pred_boxes[:, 2], pred_boxes[:, 3],
                      gt_boxes, box_weights, heat, rwc, anyc, ml)
    return out[0, 0]


# speedup vs baseline: 1.3568x; 1.0001x over previous
import functools

import jax
import jax.numpy as jnp
from jax import lax
from jax.experimental import pallas as pl
from jax.experimental.pallas import tpu as pltpu
from jax.experimental.pallas import tpu_sc as plsc

_N = 20000
_C = 80
_M = 64
_IMG = 640.0
_RADIUS = 0.75
_SIGMA = 0.5
_MARGIN = 8.0
_AREA_T = 0.8
_ALPHA = 0.75
_GAMMA = 2.0

def _axis_index(name, dim):
    return lax.axis_index(name)


_NW = 32
_LANES = 16
_CHUNKS = _N // _LANES
_BASE_CHUNKS = _CHUNKS // _NW
_EXTRA = _CHUNKS - _BASE_CHUNKS * _NW
_MAXC = _BASE_CHUNKS + 1
_LOC = _MAXC * _LANES


def _sc_assign_body(xs_hbm, ys_hbm, gx1_hbm, gy1_hbm, gx2_hbm, gy2_hbm, w_hbm,
                    heat_hbm, rwc_hbm, anyc_hbm,
                    xs_v, ys_v, gx1_v, gy1_v, gx2_v, gy2_v, w_v,
                    heat_v, rwc_v, anyc_v):
    cid = _axis_index("c", 0)
    sid = _axis_index("s", 1)
    wid = sid * 2 + cid
    cnt = _BASE_CHUNKS + jnp.where(wid >= _NW - _EXTRA, 1, 0)
    start = _BASE_CHUNKS * wid + jnp.maximum(wid - (_NW - _EXTRA), 0)
    row0 = start * _LANES

    pltpu.sync_copy(xs_hbm.at[pl.ds(row0, _LOC)], xs_v)
    pltpu.sync_copy(ys_hbm.at[pl.ds(row0, _LOC)], ys_v)
    pltpu.sync_copy(gx1_hbm, gx1_v)
    pltpu.sync_copy(gy1_hbm, gy1_v)
    pltpu.sync_copy(gx2_hbm, gx2_v)
    pltpu.sync_copy(gy2_hbm, gy2_v)
    pltpu.sync_copy(w_hbm, w_v)

    bx1, by1, bx2, by2, bcx, bcy, bisx, bisy, bw = [], [], [], [], [], [], [], [], []
    for j in range(_M // _LANES):
        s = pl.ds(j * _LANES, _LANES)
        x1 = gx1_v[s]
        y1 = gy1_v[s]
        x2 = gx2_v[s]
        y2 = gy2_v[s]
        hx = jnp.maximum((x2 - x1) * 0.5, 1.0)
        hy = jnp.maximum((y2 - y1) * 0.5, 1.0)
        bx1.append(x1)
        by1.append(y1)
        bx2.append(x2)
        by2.append(y2)
        bcx.append((x1 + x2) * 0.5)
        bcy.append((y1 + y2) * 0.5)
        bisx.append(0.7071067811865476 / (hx * _SIGMA))
        bisy.append(0.7071067811865476 / (hy * _SIGMA))
        bw.append(w_v[s])

    lim = (_RADIUS / _SIGMA) * 0.7071067811865476
    big = jnp.float32(3.0e38)

    def outer(i, carry):
        lo, hi = carry
        base = i * _LANES
        xv = xs_v[pl.ds(base, _LANES)]
        yv = ys_v[pl.ds(base, _LANES)]
        vmask = jnp.where(i < cnt, jnp.int32(-1), jnp.int32(0))
        d2m = jnp.full((_LANES,), big, jnp.float32)
        rwc = jnp.zeros((_LANES,), jnp.float32)
        loc = jnp.zeros((_LANES,), jnp.int32)
        hic = jnp.zeros((_LANES,), jnp.int32)
        for j in range(_M // _LANES):
            for k in range(_LANES):
                m = j * _LANES + k
                x1 = bx1[j][k]
                y1 = by1[j][k]
                x2 = bx2[j][k]
                y2 = by2[j][k]
                dxs = (xv - bcx[j][k]) * bisx[j][k]
                dys = (yv - bcy[j][k]) * bisy[j][k]
                ins = (xv >= x1) & (xv <= x2) & (yv >= y1) & (yv <= y2)
                d2 = dxs * dxs + dys * dys
                d2m = jnp.minimum(d2m, jnp.where(ins, d2, big))
                cand = ins & (jnp.maximum(jnp.abs(dxs), jnp.abs(dys)) <= lim)
                rwc = jnp.maximum(rwc, jnp.where(cand, bw[j][k], 0.0))
                mbit = (1 << (m % 32)) & 0xFFFFFFFF
                mbit = mbit - (1 << 32) if mbit >= (1 << 31) else mbit
                hit = jnp.where(cand, jnp.int32(mbit), 0)
                if m < 32:
                    loc = loc | hit
                else:
                    hic = hic | hit
        heat_v[pl.ds(base, _LANES)] = jnp.exp(-jnp.minimum(d2m, 100.0))
        rwc_v[pl.ds(base, _LANES)] = rwc
        return lo | (loc & vmask), hi | (hic & vmask)

    zi = jnp.zeros((_LANES,), jnp.int32)
    lo, hi = lax.fori_loop(0, _MAXC, outer, (zi, zi))
    anyc_v[pl.ds(0, _LANES)] = lo
    anyc_v[pl.ds(_LANES, _LANES)] = hi

    base_rows = _BASE_CHUNKS * _LANES
    pltpu.sync_copy(heat_v.at[pl.ds(0, base_rows)], heat_hbm.at[pl.ds(row0, base_rows)])
    pltpu.sync_copy(rwc_v.at[pl.ds(0, base_rows)], rwc_hbm.at[pl.ds(row0, base_rows)])

    @pl.when(cnt > _BASE_CHUNKS)
    def _():
        pltpu.sync_copy(heat_v.at[pl.ds(base_rows, _LANES)],
                        heat_hbm.at[pl.ds(row0 + base_rows, _LANES)])
        pltpu.sync_copy(rwc_v.at[pl.ds(base_rows, _LANES)],
                        rwc_hbm.at[pl.ds(row0 + base_rows, _LANES)])

    pltpu.sync_copy(anyc_v, anyc_hbm.at[wid])


def _sc_assign(xs, ys, gx1, gy1, gx2, gy2, w):
    call = functools.partial(
        pl.kernel,
        out_type=(
            jax.ShapeDtypeStruct((_N,), jnp.float32),
            jax.ShapeDtypeStruct((_N,), jnp.float32),
            jax.ShapeDtypeStruct((_NW, 2 * _LANES), jnp.int32),
        ),
        mesh=plsc.VectorSubcoreMesh(core_axis_name="c", subcore_axis_name="s"),
        scratch_types=[
            pltpu.VMEM((_LOC,), jnp.float32),
            pltpu.VMEM((_LOC,), jnp.float32),
            pltpu.VMEM((_M,), jnp.float32),
            pltpu.VMEM((_M,), jnp.float32),
            pltpu.VMEM((_M,), jnp.float32),
            pltpu.VMEM((_M,), jnp.float32),
            pltpu.VMEM((_M,), jnp.float32),
            pltpu.VMEM((_LOC,), jnp.float32),
            pltpu.VMEM((_LOC,), jnp.float32),
            pltpu.VMEM((2 * _LANES,), jnp.int32),
        ],
    )(_sc_assign_body)
    return call(xs, ys, gx1, gy1, gx2, gy2, w)


def _margin_body(scores_ref, out_ref):
    mx = jnp.max(scores_ref[...], axis=1, keepdims=True)
    out_ref[0] = jax.nn.relu(mx - _MARGIN)


def _tc_margin(pred_scores):
    nb = 4
    blk = _N // nb
    out = pl.pallas_call(
        _margin_body,
        grid=(nb,),
        in_specs=[pl.BlockSpec((blk, _C), lambda i: (i, 0))],
        out_specs=pl.BlockSpec((1, blk, 1), lambda i: (i, 0, 0)),
        out_shape=jax.ShapeDtypeStruct((nb, blk, 1), jnp.float32),
    )(pred_scores)
    return out.reshape(_N)


_NB2 = 10
_BLK2 = _N // _NB2


def _combine_body(xs_ref, ys_ref, obj_ref, px1_ref, py1_ref, px2_ref, py2_ref,
                  gt_ref, w_ref, heat_ref, rwc_ref, anyc_ref, ml_ref,
                  out_ref, rw_ref):
    lohi = anyc_ref[...]
    blo = lohi[:, :_LANES]
    bhi = lohi[:, _LANES:]
    sh = lax.broadcasted_iota(jnp.int32, (1, 1, 32), 2)
    any_lo = jnp.max(jnp.max((blo[:, :, None] >> sh) & 1, axis=0), axis=0)
    any_hi = jnp.max(jnp.max((bhi[:, :, None] >> sh) & 1, axis=0), axis=0)
    anyc = jnp.concatenate([any_lo, any_hi]).astype(jnp.float32)
    wfb = jnp.where(anyc > 0.0, 0.0, w_ref[...])[:, None]
    has_fb = jnp.sum(wfb) > 0.0

    b = gt_ref[...]
    x1 = b[:, 0][:, None]
    y1 = b[:, 1][:, None]
    x2 = b[:, 2][:, None]
    y2 = b[:, 3][:, None]

    @pl.when(has_fb)
    def _():
        for c in range(_NB2):
            s = pl.ds(c * _BLK2, _BLK2)
            xr = xs_ref[s][None, :]
            yr = ys_ref[s][None, :]
            inside = (xr >= x1) & (xr <= x2) & (yr >= y1) & (yr <= y2)
            fb = jnp.max(jnp.where(inside, wfb, 0.0), axis=0)
            rw_ref[s] = jnp.maximum(rwc_ref[s], fb)

    @pl.when(jnp.logical_not(has_fb))
    def _():
        rw_ref[...] = rwc_ref[...]

    rw = rw_ref[...]

    heat_t = heat_ref[...]
    l = obj_ref[...]
    prob = jax.nn.sigmoid(l)
    vfl_w = _ALPHA * prob * prob * (1.0 - heat_t) + heat_t
    bce = jnp.maximum(l, 0.0) - l * heat_t + jnp.log1p(jnp.exp(-jnp.abs(l)))
    vfl_sum = jnp.sum(bce * vfl_w)

    m_num = jnp.sum(ml_ref[...] * rw)
    m_den = jnp.maximum(jnp.sum(rw), 1e-6)

    bx1 = jnp.clip(px1_ref[...], 0.0, _IMG)
    by1 = jnp.clip(py1_ref[...], 0.0, _IMG)
    bx2 = jnp.clip(px2_ref[...], 0.0, _IMG)
    by2 = jnp.clip(py2_ref[...], 0.0, _IMG)
    wd = jnp.maximum(bx2 - bx1, 0.0)
    ht = jnp.maximum(by2 - by1, 0.0)
    area_ratio = wd * ht * (1.0 / (_IMG * _IMG))
    edge = ((bx1 <= 1.0).astype(jnp.float32) + (by1 <= 1.0).astype(jnp.float32) +
            (bx2 >= _IMG - 1.0).astype(jnp.float32) +
            (by2 >= _IMG - 1.0).astype(jnp.float32)) * 0.25
    pen_sum = jnp.sum(jnp.maximum(area_ratio - _AREA_T, 0.0) * (1.0 + edge))

    total = (vfl_sum * (1.0 / _N) + m_num / m_den + pen_sum * (1.0 / _N))
    out_ref[...] = jnp.full((1, 1), total, jnp.float32)


def _tc_combine(xs, ys, obj, px1, py1, px2, py2, gt, w, heat, rwc, anyc, ml):
    return pl.pallas_call(
        _combine_body,
        out_shape=jax.ShapeDtypeStruct((1, 1), jnp.float32),
        scratch_shapes=[pltpu.VMEM((_N,), jnp.float32)],
    )(xs, ys, obj, px1, py1, px2, py2, gt, w, heat, rwc, anyc, ml)


def kernel(pred_scores, pred_boxes, pred_objectness, anchor_points, gt_boxes, gt_labels, box_weights):
    xs = anchor_points[:, 0]
    ys = anchor_points[:, 1]
    gx1 = gt_boxes[:, 0]
    gy1 = gt_boxes[:, 1]
    gx2 = gt_boxes[:, 2]
    gy2 = gt_boxes[:, 3]
    ml = _tc_margin(pred_scores)
    heat, rwc, anyc = _sc_assign(xs, ys, gx1, gy1, gx2, gy2, box_weights)
    out = _tc_combine(xs, ys, pred_objectness,
                      pred_boxes[:, 0], pred_boxes[:, 1],
                      pred_boxes[:, 2], pred_boxes[:, 3],
                      gt_boxes, box_weights, heat, rwc, anyc, ml)
    return out[0, 0]
